# baseline (speedup 1.0000x reference)
import functools

import jax
import jax.numpy as jnp
from jax import lax
from jax.experimental import pallas as pl
from jax.experimental.pallas import tpu as pltpu
from jax.experimental.pallas import tpu_sc as plsc

N_NODES = 10000
N_EDGES = 320000
D = 128
NC = 2
NS = 16
NW = NC * NS
EPW = N_EDGES // NW
B = 80
NB = EPW // B
NPAD = 10240

_mesh = plsc.VectorSubcoreMesh(core_axis_name="c", subcore_axis_name="s")

_GDN = lax.GatherDimensionNumbers(offset_dims=(), collapsed_slice_dims=(0,),
                                  start_index_map=(0,))


def _perm(x, idx):
    return lax.gather(x, idx, _GDN, (1,),
                      mode=lax.GatherScatterMode.PROMISE_IN_BOUNDS)


@functools.partial(
    pl.kernel,
    out_type=(
        jax.ShapeDtypeStruct((NW, NB, B), jnp.float32),
        jax.ShapeDtypeStruct((NPAD,), jnp.float32),
        jax.ShapeDtypeStruct((NPAD,), jnp.float32),
    ),
    mesh=_mesh,
    scratch_types=[
        pltpu.VMEM((NB, B), jnp.int32),
        pltpu.VMEM((NB, B), jnp.int32),
        pltpu.VMEM((B, D), jnp.float32),
        pltpu.VMEM((B, D), jnp.float32),
        pltpu.VMEM((B, D), jnp.float32),
        pltpu.VMEM((B, D), jnp.float32),
        pltpu.VMEM((NB, B), jnp.float32),
        pltpu.VMEM_SHARED((NPAD,), jnp.float32),
        pltpu.VMEM((NPAD // NS,), jnp.float32),
        pltpu.SemaphoreType.DMA,
        pltpu.SemaphoreType.DMA,
        pltpu.SemaphoreType.DMA,
    ],
)
def _k1(eu_hbm, ev_hbm, srci_hbm, dsti_hbm, ag_hbm, sp0_hbm, sp1_hbm,
        sidx, didx, u0, u1, v0, v1, ag_all, seg_sh, zbuf,
        sem_r0, sem_r1, sem_sc):
    cid = lax.axis_index("c")
    sid = lax.axis_index("s")
    wid = cid * NS + sid
    zslice = NPAD // NS
    for i in range(zslice // 16):
        zbuf[pl.ds(i * 16, 16)] = jnp.zeros((16,), jnp.float32)
    pltpu.sync_copy(zbuf, seg_sh.at[pl.ds(sid * zslice, zslice)])
    pltpu.sync_copy(srci_hbm.at[wid], sidx)
    pltpu.sync_copy(dsti_hbm.at[wid], didx)
    plsc.subcore_barrier()

    lanes = lax.iota(jnp.int32, 16)
    perm_idx = [(lanes ^ sh).reshape(16, 1) for sh in (8, 4, 2, 1)]
    slots = ((u0, v0, sem_r0), (u1, v1, sem_r1))

    def issue(b, slot):
        u, v, sem = slots[slot]
        pltpu.async_copy(eu_hbm.at[sidx.at[b]], u, sem)
        pltpu.async_copy(ev_hbm.at[didx.at[b]], v, sem)

    def process(b, slot):
        u, v, sem = slots[slot]
        pltpu.make_async_copy(eu_hbm.at[sidx.at[b]], u, sem).wait()
        pltpu.make_async_copy(ev_hbm.at[didx.at[b]], v, sem).wait()

        def group_body(gg, c2):
            def edge_body(ee, dotv):
                e = gg * 16 + ee
                p = [u[e, pl.ds(kk * 16, 16)] * v[e, pl.ds(kk * 16, 16)]
                     for kk in range(D // 16)]
                while len(p) > 1:
                    p = [a + b for a, b in zip(p[::2], p[1::2])]
                acc = p[0]
                for pi in perm_idx:
                    acc = acc + _perm(acc, pi)
                return jnp.where(lanes == ee, acc, dotv)

            dotv = lax.fori_loop(0, 16, edge_body,
                                 jnp.zeros((16,), jnp.float32), unroll=8)
            ag = 1.0 / (1.0 + jnp.exp(-jnp.exp(dotv)))
            ag_all[b, pl.ds(gg * 16, 16)] = ag
            return c2

        lax.fori_loop(0, B // 16, group_body, 0)
        pltpu.async_copy(ag_all.at[b], seg_sh.at[sidx.at[b]], sem_sc,
                         add=True)

    issue(0, 0)

    def pair_body(j, carry):
        b = 2 * j
        issue(b + 1, 1)
        process(b, 0)
        issue(b + 2, 0)
        process(b + 1, 1)
        return carry

    lax.fori_loop(0, (NB - 1) // 2, pair_body, 0)
    process(NB - 1, 0)

    def drain_body(b, carry):
        pltpu.make_async_copy(ag_all.at[0], seg_sh.at[sidx.at[0]],
                              sem_sc).wait()
        return carry

    lax.fori_loop(0, NB, drain_body, 0)
    pltpu.sync_copy(ag_all, ag_hbm.at[wid])
    plsc.subcore_barrier()

    @pl.when(sid == 0)
    def _():
        @pl.when(cid == 0)
        def _():
            pltpu.sync_copy(seg_sh, sp0_hbm)

        @pl.when(cid == 1)
        def _():
            pltpu.sync_copy(seg_sh, sp1_hbm)


_K2R = 4


@functools.partial(
    pl.kernel,
    out_type=jax.ShapeDtypeStruct((NW, NB, B), jnp.float32),
    mesh=_mesh,
    scratch_types=[
        pltpu.VMEM((NB, B), jnp.int32),
        pltpu.VMEM((NB, B), jnp.float32),
        pltpu.VMEM((NB, B), jnp.float32),
        pltpu.VMEM_SHARED((NPAD,), jnp.float32),
        pltpu.VMEM((NPAD // NS,), jnp.float32),
        pltpu.VMEM((NPAD // NS,), jnp.float32),
        [pltpu.VMEM((B,), jnp.float32) for _ in range(_K2R)],
        [pltpu.SemaphoreType.DMA for _ in range(_K2R)],
    ],
)
def _k2(ag_hbm, sp0_hbm, sp1_hbm, srci_hbm, c_hbm,
        sidx, ag_all, c_all, s_sh, t0, t1, s_ring, sems):
    cid = lax.axis_index("c")
    sid = lax.axis_index("s")
    wid = cid * NS + sid
    zslice = NPAD // NS
    pltpu.sync_copy(sp0_hbm.at[pl.ds(sid * zslice, zslice)], t0)
    pltpu.sync_copy(sp1_hbm.at[pl.ds(sid * zslice, zslice)], t1)
    for i in range(zslice // 16):
        sl = pl.ds(i * 16, 16)
        t0[sl] = t0[sl] + t1[sl]
    pltpu.sync_copy(t0, s_sh.at[pl.ds(sid * zslice, zslice)])
    pltpu.sync_copy(srci_hbm.at[wid], sidx)
    pltpu.sync_copy(ag_hbm.at[wid], ag_all)
    plsc.subcore_barrier()

    def issue(b, slot):
        @pl.when(b < NB)
        def _():
            pltpu.async_copy(s_sh.at[sidx.at[b]], s_ring[slot], sems[slot])

    def process(b, slot):
        s = s_ring[slot]
        pltpu.make_async_copy(s_sh.at[sidx.at[b]], s, sems[slot]).wait()

        def group_body(gg, c2):
            sl = pl.ds(gg * 16, 16)
            c = jnp.minimum(jnp.maximum(ag_all[b, sl] * 5.0 / s[sl], 0.0),
                            1.0)
            c_all[b, sl] = c
            return c2

        lax.fori_loop(0, B // 16, group_body, 0)

    for k in range(_K2R):
        issue(k, k)

    def ring_body(j, carry):
        for k in range(_K2R):
            b = _K2R * j + k
            process(b, k)
            issue(b + _K2R, k)
        return carry

    lax.fori_loop(0, NB // _K2R, ring_body, 0)
    for k in range(NB - _K2R * (NB // _K2R)):
        process(_K2R * (NB // _K2R) + k, k)
    pltpu.sync_copy(c_all, c_hbm.at[wid])


def kernel(Eu, Ev, edge_index, g):
    g32 = g.astype(jnp.float32)
    eu_s = (Eu.astype(jnp.float32) * (g32 * g32)).astype(jnp.float32)
    ev = Ev.astype(jnp.float32)
    ei = edge_index.astype(jnp.int32)
    src3 = ei[0].reshape(NW, NB, B)
    dst3 = ei[1].reshape(NW, NB, B)
    ag, sp0, sp1 = _k1(eu_s, ev, src3, dst3)
    c3 = _k2(ag, sp0, sp1, src3)
    return c3.reshape(N_EDGES)


# speedup vs baseline: 1.1064x; 1.1064x over previous
import functools

import jax
import jax.numpy as jnp
from jax import lax
from jax.experimental import pallas as pl
from jax.experimental.pallas import tpu as pltpu
from jax.experimental.pallas import tpu_sc as plsc

N_NODES = 10000
N_EDGES = 320000
D = 128
NC = 2
NS = 16
NW = NC * NS
EPW = N_EDGES // NW
B = 80
NB = EPW // B
NPAD = 10240
_R = 4

_mesh = plsc.VectorSubcoreMesh(core_axis_name="c", subcore_axis_name="s")

_GDN = lax.GatherDimensionNumbers(offset_dims=(), collapsed_slice_dims=(0,),
                                  start_index_map=(0,))


def _perm(x, idx):
    return lax.gather(x, idx, _GDN, (1,),
                      mode=lax.GatherScatterMode.PROMISE_IN_BOUNDS)


@functools.partial(
    pl.kernel,
    out_type=(
        jax.ShapeDtypeStruct((NW, NB, B), jnp.float32),
        jax.ShapeDtypeStruct((NPAD,), jnp.float32),
        jax.ShapeDtypeStruct((NPAD,), jnp.float32),
    ),
    mesh=_mesh,
    scratch_types=[
        pltpu.VMEM((NB, B), jnp.int32),
        pltpu.VMEM((NB, B), jnp.int32),
        pltpu.VMEM((B, D), jnp.float32),
        pltpu.VMEM((B, D), jnp.float32),
        pltpu.VMEM((B, D), jnp.float32),
        pltpu.VMEM((B, D), jnp.float32),
        pltpu.VMEM((NB, B), jnp.float32),
        pltpu.VMEM((NB, B), jnp.float32),
        pltpu.VMEM_SHARED((NPAD,), jnp.float32),
        pltpu.VMEM((NPAD // NS,), jnp.float32),
        pltpu.VMEM((NPAD // NS,), jnp.int32),
        [pltpu.VMEM((B,), jnp.float32) for _ in range(_R)],
        pltpu.SemaphoreType.DMA,
        pltpu.SemaphoreType.DMA,
        pltpu.SemaphoreType.DMA,
        [pltpu.SemaphoreType.DMA for _ in range(_R)],
        pltpu.SemaphoreType.REGULAR,
    ],
)
def _k(eu_hbm, ev_hbm, srci_hbm, dsti_hbm, c_hbm, sp0_hbm, sp1_hbm,
       sidx, didx, u0, u1, v0, v1, ag_all, c_all, seg_sh, zbuf, iota_v,
       s_ring, sem_r0, sem_r1, sem_sc, ring_sems, xsem):
    cid = lax.axis_index("c")
    sid = lax.axis_index("s")
    wid = cid * NS + sid
    zslice = NPAD // NS
    lanes = lax.iota(jnp.int32, 16)
    for i in range(zslice // 16):
        zbuf[pl.ds(i * 16, 16)] = jnp.zeros((16,), jnp.float32)
        iota_v[pl.ds(i * 16, 16)] = lanes + (sid * zslice + i * 16)
    pltpu.sync_copy(zbuf, seg_sh.at[pl.ds(sid * zslice, zslice)])
    pltpu.sync_copy(srci_hbm.at[wid], sidx)
    pltpu.sync_copy(dsti_hbm.at[wid], didx)
    plsc.subcore_barrier()

    perm_idx = [(lanes ^ sh).reshape(16, 1) for sh in (8, 4, 2, 1)]
    slots = ((u0, v0, sem_r0), (u1, v1, sem_r1))

    def issue(b, slot):
        u, v, sem = slots[slot]
        pltpu.async_copy(eu_hbm.at[sidx.at[b]], u, sem)
        pltpu.async_copy(ev_hbm.at[didx.at[b]], v, sem)

    def process(b, slot):
        u, v, sem = slots[slot]
        pltpu.make_async_copy(eu_hbm.at[sidx.at[b]], u, sem).wait()
        pltpu.make_async_copy(ev_hbm.at[didx.at[b]], v, sem).wait()

        def group_body(gg, c2):
            def edge_body(ee, dotv):
                e = gg * 16 + ee
                acc = u[e, pl.ds(0, 16)] * v[e, pl.ds(0, 16)]
                for kk in range(1, D // 16):
                    acc = acc + (u[e, pl.ds(kk * 16, 16)]
                                 * v[e, pl.ds(kk * 16, 16)])
                for pi in perm_idx:
                    acc = acc + _perm(acc, pi)
                return jnp.where(lanes == ee, acc, dotv)

            dotv = lax.fori_loop(0, 16, edge_body,
                                 jnp.zeros((16,), jnp.float32), unroll=4)
            ag = 1.0 / (1.0 + jnp.exp(-jnp.exp(dotv)))
            ag_all[b, pl.ds(gg * 16, 16)] = ag
            return c2

        lax.fori_loop(0, B // 16, group_body, 0)
        pltpu.async_copy(ag_all.at[b], seg_sh.at[sidx.at[b]], sem_sc,
                         add=True)

    issue(0, 0)

    def pair_body(j, carry):
        b = 2 * j
        issue(b + 1, 1)
        process(b, 0)
        issue(b + 2, 0)
        process(b + 1, 1)
        return carry

    lax.fori_loop(0, (NB - 1) // 2, pair_body, 0)
    process(NB - 1, 0)

    def drain_body(b, carry):
        pltpu.make_async_copy(ag_all.at[0], seg_sh.at[sidx.at[0]],
                              sem_sc).wait()
        return carry

    lax.fori_loop(0, NB, drain_body, 0)
    plsc.subcore_barrier()

    @pl.when(sid == 0)
    def _():
        @pl.when(cid == 0)
        def _():
            pltpu.sync_copy(seg_sh, sp0_hbm)

        @pl.when(cid == 1)
        def _():
            pltpu.sync_copy(seg_sh, sp1_hbm)

        pltpu.semaphore_signal(xsem, 1, core_index=1 - cid)
        pl.semaphore_wait(xsem, 1)

    plsc.subcore_barrier()

    @pl.when(cid == 0)
    def _():
        pltpu.sync_copy(sp1_hbm.at[pl.ds(sid * zslice, zslice)], zbuf)

    @pl.when(cid == 1)
    def _():
        pltpu.sync_copy(sp0_hbm.at[pl.ds(sid * zslice, zslice)], zbuf)

    pltpu.sync_copy(zbuf, seg_sh.at[iota_v], add=True)
    plsc.subcore_barrier()

    def issue2(b, slot):
        @pl.when(b < NB)
        def _():
            pltpu.async_copy(seg_sh.at[sidx.at[b]], s_ring[slot],
                             ring_sems[slot])

    def process2(b, slot):
        s = s_ring[slot]
        pltpu.make_async_copy(seg_sh.at[sidx.at[b]], s,
                              ring_sems[slot]).wait()

        def group_body(gg, c2):
            sl = pl.ds(gg * 16, 16)
            c = jnp.minimum(jnp.maximum(ag_all[b, sl] * 5.0 / s[sl], 0.0),
                            1.0)
            c_all[b, sl] = c
            return c2

        lax.fori_loop(0, B // 16, group_body, 0)

    for k in range(_R):
        issue2(k, k)

    def ring_body(j, carry):
        for k in range(_R):
            b = _R * j + k
            process2(b, k)
            issue2(b + _R, k)
        return carry

    lax.fori_loop(0, NB // _R, ring_body, 0)
    for k in range(NB - _R * (NB // _R)):
        process2(_R * (NB // _R) + k, k)
    pltpu.sync_copy(c_all, c_hbm.at[wid])


def kernel(Eu, Ev, edge_index, g):
    g32 = g.astype(jnp.float32)
    eu_s = (Eu.astype(jnp.float32) * (g32 * g32)).astype(jnp.float32)
    ev = Ev.astype(jnp.float32)
    ei = edge_index.astype(jnp.int32)
    src3 = ei[0].reshape(NW, NB, B)
    dst3 = ei[1].reshape(NW, NB, B)
    c3, _, _ = _k(eu_s, ev, src3, dst3)
    return c3.reshape(N_EDGES)


# speedup vs baseline: 1.1934x; 1.0787x over previous
import functools

import jax
import jax.numpy as jnp
from jax import lax
from jax.experimental import pallas as pl
from jax.experimental.pallas import tpu as pltpu
from jax.experimental.pallas import tpu_sc as plsc

N_NODES = 10000
N_EDGES = 320000
D = 128
NC = 2
NS = 16
NW = NC * NS
EPW = N_EDGES // NW
B = 80
NB = EPW // B
NPAD = 10240
_R = 4

_mesh = plsc.VectorSubcoreMesh(core_axis_name="c", subcore_axis_name="s")

_GDN = lax.GatherDimensionNumbers(offset_dims=(), collapsed_slice_dims=(0,),
                                  start_index_map=(0,))


def _perm(x, idx):
    return lax.gather(x, idx, _GDN, (1,),
                      mode=lax.GatherScatterMode.PROMISE_IN_BOUNDS)


@functools.partial(
    pl.kernel,
    out_type=(
        jax.ShapeDtypeStruct((NW, NB, B), jnp.float32),
        jax.ShapeDtypeStruct((NPAD,), jnp.float32),
        jax.ShapeDtypeStruct((NPAD,), jnp.float32),
    ),
    mesh=_mesh,
    scratch_types=[
        pltpu.VMEM((D,), jnp.float32),
        pltpu.VMEM((NB, B), jnp.int32),
        pltpu.VMEM((NB, B), jnp.int32),
        pltpu.VMEM((B, D), jnp.float32),
        pltpu.VMEM((B, D), jnp.float32),
        pltpu.VMEM((B, D), jnp.float32),
        pltpu.VMEM((B, D), jnp.float32),
        pltpu.VMEM((NB, B), jnp.float32),
        pltpu.VMEM((NB, B), jnp.float32),
        pltpu.VMEM_SHARED((NPAD,), jnp.float32),
        pltpu.VMEM((NPAD // NS,), jnp.float32),
        pltpu.VMEM((NPAD // NS,), jnp.int32),
        [pltpu.VMEM((B,), jnp.float32) for _ in range(_R)],
        pltpu.SemaphoreType.DMA,
        pltpu.SemaphoreType.DMA,
        pltpu.SemaphoreType.DMA,
        [pltpu.SemaphoreType.DMA for _ in range(_R)],
        pltpu.SemaphoreType.REGULAR,
    ],
)
def _k(eu_hbm, ev_hbm, ei_hbm, g_hbm, c_hbm, sp0_hbm, sp1_hbm,
       g_v, sidx, didx, u0, u1, v0, v1, ag_all, c_all, seg_sh, zbuf, iota_v,
       s_ring, sem_r0, sem_r1, sem_sc, ring_sems, xsem):
    cid = lax.axis_index("c")
    sid = lax.axis_index("s")
    wid = cid * NS + sid
    zslice = NPAD // NS
    lanes = lax.iota(jnp.int32, 16)
    for i in range(zslice // 16):
        zbuf[pl.ds(i * 16, 16)] = jnp.zeros((16,), jnp.float32)
        iota_v[pl.ds(i * 16, 16)] = lanes + (sid * zslice + i * 16)
    pltpu.sync_copy(zbuf, seg_sh.at[pl.ds(sid * zslice, zslice)])
    pltpu.sync_copy(ei_hbm.at[0, wid], sidx)
    pltpu.sync_copy(ei_hbm.at[1, wid], didx)
    pltpu.sync_copy(g_hbm, g_v)
    plsc.subcore_barrier()
    g2 = [g_v[pl.ds(kk * 16, 16)] * g_v[pl.ds(kk * 16, 16)]
          for kk in range(D // 16)]

    perm_idx = [(lanes ^ sh).reshape(16, 1) for sh in (8, 4, 2, 1)]
    slots = ((u0, v0, sem_r0), (u1, v1, sem_r1))

    def issue(b, slot):
        u, v, sem = slots[slot]
        pltpu.async_copy(eu_hbm.at[sidx.at[b]], u, sem)
        pltpu.async_copy(ev_hbm.at[didx.at[b]], v, sem)

    def process(b, slot):
        u, v, sem = slots[slot]
        pltpu.make_async_copy(eu_hbm.at[sidx.at[b]], u, sem).wait()
        pltpu.make_async_copy(ev_hbm.at[didx.at[b]], v, sem).wait()

        def group_body(gg, c2):
            def edge_body(ee, dotv):
                e = gg * 16 + ee
                acc = (u[e, pl.ds(0, 16)] * g2[0]) * v[e, pl.ds(0, 16)]
                for kk in range(1, D // 16):
                    acc = acc + ((u[e, pl.ds(kk * 16, 16)] * g2[kk])
                                 * v[e, pl.ds(kk * 16, 16)])
                for pi in perm_idx:
                    acc = acc + _perm(acc, pi)
                return jnp.where(lanes == ee, acc, dotv)

            dotv = lax.fori_loop(0, 16, edge_body,
                                 jnp.zeros((16,), jnp.float32), unroll=4)
            ag = 1.0 / (1.0 + jnp.exp(-jnp.exp(dotv)))
            ag_all[b, pl.ds(gg * 16, 16)] = ag
            return c2

        lax.fori_loop(0, B // 16, group_body, 0)
        pltpu.async_copy(ag_all.at[b], seg_sh.at[sidx.at[b]], sem_sc,
                         add=True)

    issue(0, 0)

    def pair_body(j, carry):
        b = 2 * j
        issue(b + 1, 1)
        process(b, 0)
        issue(b + 2, 0)
        process(b + 1, 1)
        return carry

    lax.fori_loop(0, (NB - 1) // 2, pair_body, 0)
    process(NB - 1, 0)

    def drain_body(b, carry):
        pltpu.make_async_copy(ag_all.at[0], seg_sh.at[sidx.at[0]],
                              sem_sc).wait()
        return carry

    lax.fori_loop(0, NB, drain_body, 0)
    plsc.subcore_barrier()

    @pl.when(sid == 0)
    def _():
        @pl.when(cid == 0)
        def _():
            pltpu.sync_copy(seg_sh, sp0_hbm)

        @pl.when(cid == 1)
        def _():
            pltpu.sync_copy(seg_sh, sp1_hbm)

        pltpu.semaphore_signal(xsem, 1, core_index=1 - cid)
        pl.semaphore_wait(xsem, 1)

    plsc.subcore_barrier()

    @pl.when(cid == 0)
    def _():
        pltpu.sync_copy(sp1_hbm.at[pl.ds(sid * zslice, zslice)], zbuf)

    @pl.when(cid == 1)
    def _():
        pltpu.sync_copy(sp0_hbm.at[pl.ds(sid * zslice, zslice)], zbuf)

    pltpu.sync_copy(zbuf, seg_sh.at[iota_v], add=True)
    plsc.subcore_barrier()

    def issue2(b, slot):
        @pl.when(b < NB)
        def _():
            pltpu.async_copy(seg_sh.at[sidx.at[b]], s_ring[slot],
                             ring_sems[slot])

    def process2(b, slot):
        s = s_ring[slot]
        pltpu.make_async_copy(seg_sh.at[sidx.at[b]], s,
                              ring_sems[slot]).wait()

        def group_body(gg, c2):
            sl = pl.ds(gg * 16, 16)
            c = jnp.minimum(jnp.maximum(ag_all[b, sl] * 5.0 / s[sl], 0.0),
                            1.0)
            c_all[b, sl] = c
            return c2

        lax.fori_loop(0, B // 16, group_body, 0)

    for k in range(_R):
        issue2(k, k)

    def ring_body(j, carry):
        for k in range(_R):
            b = _R * j + k
            process2(b, k)
            issue2(b + _R, k)
        return carry

    lax.fori_loop(0, NB // _R, ring_body, 0)
    for k in range(NB - _R * (NB // _R)):
        process2(_R * (NB // _R) + k, k)
    pltpu.sync_copy(c_all, c_hbm.at[wid])


def kernel(Eu, Ev, edge_index, g):
    ei4 = edge_index.astype(jnp.int32).reshape(2, NW, NB, B)
    g1 = g.astype(jnp.float32).reshape(D)
    c3, _, _ = _k(Eu.astype(jnp.float32), Ev.astype(jnp.float32), ei4, g1)
    return c3.reshape(N_EDGES)
